# baseline (speedup 1.0000x reference)
import functools

import jax
import jax.numpy as jnp
from jax import lax
from jax.experimental import pallas as pl
from jax.experimental.pallas import tpu as pltpu
from jax.experimental.pallas import tpu_sc as plsc

N_USER = 2000
N_ITEM = 8000
N = N_USER + N_ITEM
E = 160000
D_FEAT = 512
D = 256
DH = D // 2
NC = 2
NS = 16
CHUNK = 128
CH_PER_TILE = 80
E_PAD = NS * CH_PER_TILE * CHUNK
ACC_ROWS = 10240
ZROWS = ACC_ROWS // NS
OROWS = ACC_ROWS // NS
RB = 1000


def _leaky(v):
    return jnp.where(v >= 0, v, 0.01 * v)


def _prep_body(pref_ref, feat_ref, w_ref, b_ref, out_ref):
    i = pl.program_id(0)

    @pl.when(i < 2)
    def _():
        v = pref_ref[...]
        nrm = jnp.sqrt(jnp.sum(v * v, axis=1, keepdims=True))
        out_ref[...] = v / jnp.maximum(nrm, 1e-12)

    @pl.when(i >= 2)
    def _():
        v = jnp.dot(feat_ref[...], w_ref[...],
                    preferred_element_type=jnp.float32) + b_ref[...]
        nrm = jnp.sqrt(jnp.sum(v * v, axis=1, keepdims=True))
        out_ref[...] = v / jnp.maximum(nrm, 1e-12)


def _prep(pref, feat, w, b):
    return pl.pallas_call(
        _prep_body,
        grid=(N // RB,),
        in_specs=[
            pl.BlockSpec((RB, D), lambda i: (jnp.minimum(i, 1), 0)),
            pl.BlockSpec((RB, D_FEAT), lambda i: (jnp.maximum(i - 2, 0), 0)),
            pl.BlockSpec((D_FEAT, D), lambda i: (0, 0)),
            pl.BlockSpec((1, D), lambda i: (0, 0)),
        ],
        out_specs=pl.BlockSpec((RB, D), lambda i: (i, 0)),
        out_shape=jax.ShapeDtypeStruct((N, D), jnp.float32),
    )(pref, feat, w, b)


def _pre_body(x_ref, cw_ref, lw_ref, lb_ref, id_ref, hx_ref, xhat_ref):
    xb = x_ref[...]
    hx = jnp.dot(xb, cw_ref[...], preferred_element_type=jnp.float32)
    hx_ref[0, :, :] = hx[:, :DH]
    hx_ref[1, :, :] = hx[:, DH:]
    xhat_ref[...] = _leaky(
        jnp.dot(xb, lw_ref[...], preferred_element_type=jnp.float32)
        + lb_ref[...]) + id_ref[...]


def _pre(x, cw, lw, lb, id_emb):
    return pl.pallas_call(
        _pre_body,
        grid=(N // RB,),
        in_specs=[
            pl.BlockSpec((RB, D), lambda i: (i, 0)),
            pl.BlockSpec((D, D), lambda i: (0, 0)),
            pl.BlockSpec((D, D), lambda i: (0, 0)),
            pl.BlockSpec((1, D), lambda i: (0, 0)),
            pl.BlockSpec((RB, D), lambda i: (i, 0)),
        ],
        out_specs=[
            pl.BlockSpec((2, RB, DH), lambda i: (0, i, 0)),
            pl.BlockSpec((RB, D), lambda i: (i, 0)),
        ],
        out_shape=[
            jax.ShapeDtypeStruct((2, N, DH), jnp.float32),
            jax.ShapeDtypeStruct((N, D), jnp.float32),
        ],
    )(x, cw, lw, lb, id_emb)


def _post_body(a0_ref, a1_ref, xhat_ref, gw_ref, gb_ref, out_ref):
    h = jnp.concatenate([_leaky(a0_ref[0]), _leaky(a1_ref[0])], axis=1)
    out_ref[...] = _leaky(
        jnp.dot(h, gw_ref[...], preferred_element_type=jnp.float32)
        + gb_ref[...] + xhat_ref[...])


def _post(agg, xhat, gw, gb):
    return pl.pallas_call(
        _post_body,
        grid=(N // RB,),
        in_specs=[
            pl.BlockSpec((1, RB, DH), lambda i: (0, i, 0)),
            pl.BlockSpec((1, RB, DH), lambda i: (1, i, 0)),
            pl.BlockSpec((RB, D), lambda i: (i, 0)),
            pl.BlockSpec((D, D), lambda i: (0, 0)),
            pl.BlockSpec((1, D), lambda i: (0, 0)),
        ],
        out_specs=pl.BlockSpec((RB, D), lambda i: (i, 0)),
        out_shape=jax.ShapeDtypeStruct((N, D), jnp.float32),
    )(agg, agg, xhat, gw, gb)


PH = 2
CH_PH = CH_PER_TILE // PH


def _sc_agg_body(hx_hbm, src_hbm, dst_hbm, zeros_hbm, out_hbm,
                 src_v, dst_v, r0, r1, acc, gs0, gs1, ss0, ss1):
    rows = (r0, r1)
    gs = (gs0, gs1)
    ss = (ss0, ss1)
    c = lax.axis_index("c")
    s = lax.axis_index("s")
    w = c * NS + s
    for p in range(PH):
        pltpu.sync_copy(src_hbm.at[w, pl.ds(p * CH_PH, CH_PH)], src_v)
        pltpu.sync_copy(dst_hbm.at[s, pl.ds(p * CH_PH, CH_PH)], dst_v)
        pltpu.async_copy(hx_hbm.at[src_v.at[0]], rows[0], gs[0])
        if p == 0:
            pltpu.sync_copy(zeros_hbm, acc.at[pl.ds(s * ZROWS, ZROWS)])
            plsc.subcore_barrier()
        pltpu.make_async_copy(hx_hbm.at[src_v.at[0]], rows[0], gs[0]).wait()
        pltpu.async_copy(rows[0], acc.at[dst_v.at[0]], ss[0], add=True)
        pltpu.async_copy(hx_hbm.at[src_v.at[1]], rows[1], gs[1])

        def group(g, carry):
            for b, dj in ((1, 1), (0, 2)):
                j = 2 * g + dj
                nb = 1 - b
                pltpu.make_async_copy(hx_hbm.at[src_v.at[j]], rows[b],
                                      gs[b]).wait()
                pltpu.async_copy(rows[b], acc.at[dst_v.at[j]], ss[b],
                                 add=True)
                pltpu.make_async_copy(rows[nb], acc.at[dst_v.at[j]],
                                      ss[nb]).wait()
                pltpu.async_copy(hx_hbm.at[src_v.at[j + 1]], rows[nb],
                                 gs[nb])
            return carry

        lax.fori_loop(0, (CH_PH - 2) // 2, group, 0)
        jl = CH_PH - 1
        pltpu.make_async_copy(hx_hbm.at[src_v.at[jl]], rows[1], gs[1]).wait()
        pltpu.async_copy(rows[1], acc.at[dst_v.at[jl]], ss[1], add=True)
        pltpu.make_async_copy(rows[0], acc.at[dst_v.at[0]], ss[0]).wait()
        pltpu.make_async_copy(rows[1], acc.at[dst_v.at[0]], ss[1]).wait()
    plsc.subcore_barrier()
    pltpu.sync_copy(acc.at[pl.ds(s * OROWS, OROWS)],
                    out_hbm.at[pl.ds((c * NS + s) * OROWS, OROWS)])


_sc_agg = pl.kernel(
    _sc_agg_body,
    out_type=jax.ShapeDtypeStruct((2 * ACC_ROWS, DH), jnp.float32),
    mesh=plsc.VectorSubcoreMesh(core_axis_name="c", subcore_axis_name="s"),
    scratch_types=[
        pltpu.VMEM((CH_PH, CHUNK), jnp.int32),
        pltpu.VMEM((CH_PH, CHUNK), jnp.int32),
        pltpu.VMEM((CHUNK, DH), jnp.float32),
        pltpu.VMEM((CHUNK, DH), jnp.float32),
        pltpu.VMEM_SHARED((ACC_ROWS, DH), jnp.float32),
        pltpu.SemaphoreType.DMA,
        pltpu.SemaphoreType.DMA,
        pltpu.SemaphoreType.DMA,
        pltpu.SemaphoreType.DMA,
    ],
)


def kernel(id_embedding, features, preference, mlp_W, mlp_b,
           conv1_W, lin1_W, lin1_b, g1_W, g1_b,
           conv2_W, lin2_W, lin2_b, g2_W, g2_b,
           conv3_W, lin3_W, lin3_b, g3_W, g3_b,
           edge_index):
    src = edge_index[0]
    dst = edge_index[1]
    pad = E_PAD - E
    src_p = jnp.concatenate([src, jnp.zeros((pad,), jnp.int32)])
    dst_p = jnp.concatenate([dst, jnp.full((pad,), N, jnp.int32)])
    src2 = jnp.stack([src_p, src_p + N]).reshape(NC * NS, CH_PER_TILE, CHUNK)
    dst2 = dst_p.reshape(NS, CH_PER_TILE, CHUNK)
    zeros = jnp.zeros((ZROWS, DH), jnp.float32)

    x = _prep(preference, features, mlp_W, mlp_b.reshape(1, D))
    for cw, lw, lb, gw, gb in (
            (conv1_W, lin1_W, lin1_b, g1_W, g1_b),
            (conv2_W, lin2_W, lin2_b, g2_W, g2_b),
            (conv3_W, lin3_W, lin3_b, g3_W, g3_b)):
        hx, xhat = _pre(x, cw, lw, lb.reshape(1, D), id_embedding)
        agg = _sc_agg(hx.reshape(2 * N, DH), src2, dst2, zeros)
        x = _post(agg.reshape(2, ACC_ROWS, DH), xhat, gw, gb.reshape(1, D))
    return x


# speedup vs baseline: 1.0947x; 1.0947x over previous
import jax
import jax.numpy as jnp
from jax import lax
from jax.experimental import pallas as pl
from jax.experimental.pallas import tpu as pltpu
from jax.experimental.pallas import tpu_sc as plsc

N_USER = 2000
N_ITEM = 8000
N = N_USER + N_ITEM
E = 160000
D_FEAT = 512
D = 256
DH = D // 2
NC = 2
NS = 16
CHUNK = 128
CH_PER_TILE = 80
E_PAD = NS * CH_PER_TILE * CHUNK
ACC_ROWS = 10240
ZROWS = ACC_ROWS // NS
OROWS = ACC_ROWS // NS
RB = 1000


def _leaky(v):
    return jnp.where(v >= 0, v, 0.01 * v)


def _prep_body(pref_ref, feat_ref, w_ref, b_ref, out_ref):
    i = pl.program_id(0)

    @pl.when(i < 2)
    def _():
        v = pref_ref[...]
        nrm = jnp.sqrt(jnp.sum(v * v, axis=1, keepdims=True))
        out_ref[...] = v / jnp.maximum(nrm, 1e-12)

    @pl.when(i >= 2)
    def _():
        v = jnp.dot(feat_ref[...], w_ref[...],
                    preferred_element_type=jnp.float32) + b_ref[...]
        nrm = jnp.sqrt(jnp.sum(v * v, axis=1, keepdims=True))
        out_ref[...] = v / jnp.maximum(nrm, 1e-12)


def _prep(pref, feat, w, b):
    return pl.pallas_call(
        _prep_body,
        grid=(N // RB,),
        in_specs=[
            pl.BlockSpec((RB, D), lambda i: (jnp.minimum(i, 1), 0)),
            pl.BlockSpec((RB, D_FEAT), lambda i: (jnp.maximum(i - 2, 0), 0)),
            pl.BlockSpec((D_FEAT, D), lambda i: (0, 0)),
            pl.BlockSpec((1, D), lambda i: (0, 0)),
        ],
        out_specs=pl.BlockSpec((RB, D), lambda i: (i, 0)),
        out_shape=jax.ShapeDtypeStruct((N, D), jnp.float32),
    )(pref, feat, w, b)


def _hx_body(x_ref, cw_ref, hx_ref):
    hx = jnp.dot(x_ref[...], cw_ref[...], preferred_element_type=jnp.float32)
    hx_ref[0, :, :] = hx[:, :DH]
    hx_ref[1, :, :] = hx[:, DH:]


def _hx(x, cw):
    return pl.pallas_call(
        _hx_body,
        grid=(N // RB,),
        in_specs=[
            pl.BlockSpec((RB, D), lambda i: (i, 0)),
            pl.BlockSpec((D, D), lambda i: (0, 0)),
        ],
        out_specs=pl.BlockSpec((2, RB, DH), lambda i: (0, i, 0)),
        out_shape=jax.ShapeDtypeStruct((2, N, DH), jnp.float32),
    )(x, cw)


def _xhat_body(x_ref, lw_ref, lb_ref, id_ref, xhat_ref):
    xhat_ref[...] = _leaky(
        jnp.dot(x_ref[...], lw_ref[...], preferred_element_type=jnp.float32)
        + lb_ref[...]) + id_ref[...]


def _xhat(x, lw, lb, id_emb):
    return pl.pallas_call(
        _xhat_body,
        grid=(N // RB,),
        in_specs=[
            pl.BlockSpec((RB, D), lambda i: (i, 0)),
            pl.BlockSpec((D, D), lambda i: (0, 0)),
            pl.BlockSpec((1, D), lambda i: (0, 0)),
            pl.BlockSpec((RB, D), lambda i: (i, 0)),
        ],
        out_specs=pl.BlockSpec((RB, D), lambda i: (i, 0)),
        out_shape=jax.ShapeDtypeStruct((N, D), jnp.float32),
    )(x, lw, lb, id_emb)


def _post_body(a0_ref, a1_ref, xhat_ref, gw_ref, gb_ref, out_ref):
    h = jnp.concatenate([_leaky(a0_ref[0]), _leaky(a1_ref[0])], axis=1)
    out_ref[...] = _leaky(
        jnp.dot(h, gw_ref[...], preferred_element_type=jnp.float32)
        + gb_ref[...] + xhat_ref[...])


def _post(agg, xhat, gw, gb):
    return pl.pallas_call(
        _post_body,
        grid=(N // RB,),
        in_specs=[
            pl.BlockSpec((1, RB, DH), lambda i: (0, i, 0)),
            pl.BlockSpec((1, RB, DH), lambda i: (1, i, 0)),
            pl.BlockSpec((RB, D), lambda i: (i, 0)),
            pl.BlockSpec((D, D), lambda i: (0, 0)),
            pl.BlockSpec((1, D), lambda i: (0, 0)),
        ],
        out_specs=pl.BlockSpec((RB, D), lambda i: (i, 0)),
        out_shape=jax.ShapeDtypeStruct((N, D), jnp.float32),
    )(agg, agg, xhat, gw, gb)


NBUF = 2
PH = 2
CH_PH = CH_PER_TILE // PH
NGRP = CH_PH // NBUF


def _sc_agg_body(hx_hbm, src_hbm, dst_hbm, zeros_hbm, out_hbm,
                 src_v, dst_v, r0, r1, acc, s0, s1):
    rows = (r0, r1)
    sems = (s0, s1)
    c = lax.axis_index("c")
    s = lax.axis_index("s")
    w = c * NS + s
    for p in range(PH):
        pltpu.sync_copy(src_hbm.at[w, pl.ds(p * CH_PH, CH_PH)], src_v)
        pltpu.sync_copy(dst_hbm.at[s, pl.ds(p * CH_PH, CH_PH)], dst_v)
        for b in range(NBUF):
            pltpu.async_copy(hx_hbm.at[src_v.at[b]], rows[b], sems[b])
        if p == 0:
            pltpu.sync_copy(zeros_hbm, acc.at[pl.ds(s * ZROWS, ZROWS)])
            plsc.subcore_barrier()

        def group(g, carry):
            for b in range(NBUF):
                j = g * NBUF + b
                pltpu.make_async_copy(hx_hbm.at[src_v.at[j]], rows[b],
                                      sems[b]).wait()
                pltpu.sync_copy(rows[b], acc.at[dst_v.at[j]], add=True)
                pltpu.async_copy(hx_hbm.at[src_v.at[j + NBUF]], rows[b],
                                 sems[b])
            return carry

        lax.fori_loop(0, NGRP - 1, group, 0)
        for b in range(NBUF):
            j = (NGRP - 1) * NBUF + b
            pltpu.make_async_copy(hx_hbm.at[src_v.at[j]], rows[b],
                                  sems[b]).wait()
            pltpu.sync_copy(rows[b], acc.at[dst_v.at[j]], add=True)
    plsc.subcore_barrier()
    pltpu.sync_copy(acc.at[pl.ds(s * OROWS, OROWS)],
                    out_hbm.at[pl.ds((c * NS + s) * OROWS, OROWS)])


_sc_agg = pl.kernel(
    _sc_agg_body,
    out_type=jax.ShapeDtypeStruct((2 * ACC_ROWS, DH), jnp.float32),
    mesh=plsc.VectorSubcoreMesh(core_axis_name="c", subcore_axis_name="s"),
    scratch_types=[
        pltpu.VMEM((CH_PH, CHUNK), jnp.int32),
        pltpu.VMEM((CH_PH, CHUNK), jnp.int32),
        pltpu.VMEM((CHUNK, DH), jnp.float32),
        pltpu.VMEM((CHUNK, DH), jnp.float32),
        pltpu.VMEM_SHARED((ACC_ROWS, DH), jnp.float32),
        pltpu.SemaphoreType.DMA,
        pltpu.SemaphoreType.DMA,
    ],
)


def kernel(id_embedding, features, preference, mlp_W, mlp_b,
           conv1_W, lin1_W, lin1_b, g1_W, g1_b,
           conv2_W, lin2_W, lin2_b, g2_W, g2_b,
           conv3_W, lin3_W, lin3_b, g3_W, g3_b,
           edge_index):
    src = edge_index[0]
    dst = edge_index[1]
    pad = E_PAD - E
    src_p = jnp.concatenate([src, jnp.zeros((pad,), jnp.int32)])
    dst_p = jnp.concatenate([dst, jnp.full((pad,), N, jnp.int32)])
    src2 = jnp.stack([src_p, src_p + N]).reshape(NC * NS, CH_PER_TILE, CHUNK)
    dst2 = dst_p.reshape(NS, CH_PER_TILE, CHUNK)
    zeros = jnp.zeros((ZROWS, DH), jnp.float32)

    x = _prep(preference, features, mlp_W, mlp_b.reshape(1, D))
    for cw, lw, lb, gw, gb in (
            (conv1_W, lin1_W, lin1_b, g1_W, g1_b),
            (conv2_W, lin2_W, lin2_b, g2_W, g2_b),
            (conv3_W, lin3_W, lin3_b, g3_W, g3_b)):
        hx = _hx(x, cw)
        agg = _sc_agg(hx.reshape(2 * N, DH), src2, dst2, zeros)
        xhat = _xhat(x, lw, lb.reshape(1, D), id_embedding)
        x = _post(agg.reshape(2, ACC_ROWS, DH), xhat, gw, gb.reshape(1, D))
    return x
